# Initial kernel scaffold; baseline (speedup 1.0000x reference)
import jax
import jax.numpy as jnp
from jax.experimental import pallas as pl


def kernel(x, edge_index, W1, u1, c1, b1, W2, u2, c2, b2, W3, u3, c3, b3, W4, u4, c4, b4, gamma, beta):
    raise NotImplementedError("write your pallas kernel here")


# speedup vs baseline: 21.2192x; 21.2192x over previous
import functools

import jax
import jax.numpy as jnp
from jax import lax
from jax.experimental import pallas as pl
from jax.experimental.pallas import tpu as pltpu
from jax.experimental.pallas import tpu_sc as plsc

N = 10000
E = 320000
H = 4
HID = 4
D_OUT = 128
NC = 2
NS = 16
NW = NC * NS
EPW = E // NW
CH = 80
NCHUNK = EPW // CH
GPC = CH // 16
SPAN = 632
LAST_SPAN = N - 15 * SPAN

_f32 = jnp.float32
_i32 = jnp.int32


def _mesh():
    return plsc.VectorSubcoreMesh(
        core_axis_name="c", subcore_axis_name="s", num_cores=NC, num_subcores=NS
    )


_SC_PARAMS = pltpu.CompilerParams(needs_layout_passes=False,
                                  use_tc_tiling_on_sc=False)


def _softmax4(sc):
    m = jnp.maximum(jnp.maximum(sc[0], sc[1]), jnp.maximum(sc[2], sc[3]))
    eh = [jnp.exp(sc[h] - m) for h in range(4)]
    tot = (eh[0] + eh[1]) + (eh[2] + eh[3])
    return eh, tot


def _full(v):
    return jnp.full((16,), v, _i32)


def _edge_softmax(atab_v, cvec_v, s16, d16):
    msk = s16 != d16
    sb = s16 * 4
    db = d16 * 4
    sc = [
        plsc.load_gather(atab_v, [sb + h])
        - plsc.load_gather(atab_v, [db + h])
        + cvec_v[h, :]
        for h in range(4)
    ]
    eh, tot = _softmax4(sc)
    rinv = jnp.where(msk, 1.0 / tot, 0.0)
    q = [eh[h] * rinv for h in range(4)]
    return q, msk


def _sc_zform():
    aw = 8

    @functools.partial(
        pl.kernel,
        out_type=jax.ShapeDtypeStruct((NC, N, aw), _f32),
        mesh=_mesh(),
        compiler_params=_SC_PARAMS,
        scratch_types=[
            pltpu.VMEM((EPW,), _i32),
            pltpu.VMEM((EPW,), _i32),
            pltpu.VMEM((N * 4,), _f32),
            pltpu.VMEM((4, 16), _f32),
            pltpu.VMEM((CH, 16), _f32),
            pltpu.VMEM((CH, 16), _f32),
            pltpu.VMEM((CH, aw), _f32),
            pltpu.VMEM((CH,), _i32),
            pltpu.VMEM((SPAN, aw), _f32),
            pltpu.VMEM_SHARED((N, aw), _f32),
            pltpu.SemaphoreType.DMA,
            pltpu.SemaphoreType.DMA,
        ],
    )
    def k(src_hbm, dst_hbm, atab_hbm, cvec_hbm, z_hbm, zero_hbm, out_hbm,
          src_v, dst_v, atab_v, cvec_v, zr0, zr1, stage, dstidx, bounce, acc,
          sem0, sem1):
        cc = lax.axis_index("c")
        ss = lax.axis_index("s")
        wid = cc * NS + ss
        ebase = wid * EPW
        pltpu.sync_copy(src_hbm.at[pl.ds(ebase, EPW)], src_v)
        pltpu.sync_copy(dst_hbm.at[pl.ds(ebase, EPW)], dst_v)
        pltpu.sync_copy(atab_hbm, atab_v)
        pltpu.sync_copy(cvec_hbm, cvec_v)
        pltpu.sync_copy(zero_hbm.at[pl.ds(0, CH)], stage)

        @pl.when(ss < NS - 1)
        def _():
            pltpu.sync_copy(zero_hbm.at[pl.ds(ss * SPAN, SPAN)],
                            acc.at[pl.ds(ss * SPAN, SPAN)])

        @pl.when(ss == NS - 1)
        def _():
            pltpu.sync_copy(zero_hbm.at[pl.ds(15 * SPAN, LAST_SPAN)],
                            acc.at[pl.ds(15 * SPAN, LAST_SPAN)])

        plsc.subcore_barrier()

        iota16 = lax.iota(_i32, 16)

        def issue(ci, zr, sem):
            pltpu.async_copy(z_hbm.at[src_v.at[pl.ds(ci * CH, CH)]], zr, sem)

        def drain(zr, sem):
            pltpu.make_async_copy(z_hbm.at[pl.ds(0, CH)], zr, sem).wait()

        def compute(ci, zr):
            for g in range(GPC):
                e0 = ci * CH + g * 16
                s16 = src_v[pl.ds(e0, 16)]
                d16 = dst_v[pl.ds(e0, 16)]
                dstidx[pl.ds(g * 16, 16)] = d16
                q, msk = _edge_softmax(atab_v, cvec_v, s16, d16)
                el = iota16 + g * 16
                for o in range(4):
                    acc_o = q[0] * plsc.load_gather(zr, [el, _full(o)])
                    for h in range(1, 4):
                        acc_o = acc_o + q[h] * plsc.load_gather(
                            zr, [el, _full(h * 4 + o)])
                    plsc.store_scatter(stage, [el, _full(o)], acc_o)
                plsc.store_scatter(stage, [el, _full(4)],
                                   jnp.where(msk, 1.0, 0.0))
            pltpu.sync_copy(stage, acc.at[dstidx], add=True)

        issue(0, zr0, sem0)

        def body(i, carry):
            c0 = i * 2
            issue(c0 + 1, zr1, sem1)
            drain(zr0, sem0)
            compute(c0, zr0)
            issue(c0 + 2, zr0, sem0)
            drain(zr1, sem1)
            compute(c0 + 1, zr1)
            return carry

        lax.fori_loop(0, (NCHUNK - 1) // 2, body, 0)
        drain(zr0, sem0)
        compute(NCHUNK - 1, zr0)

        plsc.subcore_barrier()

        @pl.when(ss < NS - 1)
        def _():
            pltpu.sync_copy(acc.at[pl.ds(ss * SPAN, SPAN)], bounce)
            pltpu.sync_copy(bounce, out_hbm.at[cc, pl.ds(ss * SPAN, SPAN)])

        @pl.when(ss == NS - 1)
        def _():
            pltpu.sync_copy(acc.at[pl.ds(15 * SPAN, LAST_SPAN)],
                            bounce.at[pl.ds(0, LAST_SPAN)])
            pltpu.sync_copy(bounce.at[pl.ds(0, LAST_SPAN)],
                            out_hbm.at[cc, pl.ds(15 * SPAN, LAST_SPAN)])

    return k


def _sc_l4():
    @functools.partial(
        pl.kernel,
        out_type=jax.ShapeDtypeStruct((NC, N, 16), _f32),
        mesh=_mesh(),
        compiler_params=_SC_PARAMS,
        scratch_types=[
            pltpu.VMEM((EPW,), _i32),
            pltpu.VMEM((EPW,), _i32),
            pltpu.VMEM((N * 4,), _f32),
            pltpu.VMEM((4, 16), _f32),
            pltpu.VMEM((N * 4,), _f32),
            pltpu.VMEM((CH, 16), _f32),
            pltpu.VMEM((CH,), _i32),
            pltpu.VMEM((SPAN, 16), _f32),
            pltpu.VMEM_SHARED((N, 16), _f32),
        ],
    )
    def k(src_hbm, dst_hbm, atab_hbm, cvec_hbm, htab_hbm, zero_hbm, out_hbm,
          src_v, dst_v, atab_v, cvec_v, htab_v, stage, dstidx, bounce, acc):
        cc = lax.axis_index("c")
        ss = lax.axis_index("s")
        wid = cc * NS + ss
        ebase = wid * EPW
        pltpu.sync_copy(src_hbm.at[pl.ds(ebase, EPW)], src_v)
        pltpu.sync_copy(dst_hbm.at[pl.ds(ebase, EPW)], dst_v)
        pltpu.sync_copy(atab_hbm, atab_v)
        pltpu.sync_copy(cvec_hbm, cvec_v)
        pltpu.sync_copy(htab_hbm, htab_v)

        @pl.when(ss < NS - 1)
        def _():
            pltpu.sync_copy(zero_hbm.at[pl.ds(ss * SPAN, SPAN)],
                            acc.at[pl.ds(ss * SPAN, SPAN)])

        @pl.when(ss == NS - 1)
        def _():
            pltpu.sync_copy(zero_hbm.at[pl.ds(15 * SPAN, LAST_SPAN)],
                            acc.at[pl.ds(15 * SPAN, LAST_SPAN)])

        plsc.subcore_barrier()

        iota16 = lax.iota(_i32, 16)

        def body(ci, carry):
            for g in range(GPC):
                e0 = ci * CH + g * 16
                s16 = src_v[pl.ds(e0, 16)]
                d16 = dst_v[pl.ds(e0, 16)]
                dstidx[pl.ds(g * 16, 16)] = d16
                q, _ = _edge_softmax(atab_v, cvec_v, s16, d16)
                sb = s16 * 4
                hv = [plsc.load_gather(htab_v, [sb + d]) for d in range(4)]
                el = iota16 + g * 16
                for h in range(4):
                    for d in range(4):
                        plsc.store_scatter(stage, [el, _full(h * 4 + d)],
                                           q[h] * hv[d])
            pltpu.sync_copy(stage, acc.at[dstidx], add=True)
            return carry

        lax.fori_loop(0, NCHUNK, body, 0)

        plsc.subcore_barrier()

        @pl.when(ss < NS - 1)
        def _():
            pltpu.sync_copy(acc.at[pl.ds(ss * SPAN, SPAN)], bounce)
            pltpu.sync_copy(bounce, out_hbm.at[cc, pl.ds(ss * SPAN, SPAN)])

        @pl.when(ss == NS - 1)
        def _():
            pltpu.sync_copy(acc.at[pl.ds(15 * SPAN, LAST_SPAN)],
                            bounce.at[pl.ds(0, LAST_SPAN)])
            pltpu.sync_copy(bounce.at[pl.ds(0, LAST_SPAN)],
                            out_hbm.at[cc, pl.ds(15 * SPAN, LAST_SPAN)])

    return k


def _dot(a, b):
    kdim = b.shape[0]
    out = a[:, 0:1] * b[0:1, :]
    for k in range(1, kdim):
        out = out + a[:, k:k + 1] * b[k:k + 1, :]
    return out


def _qs_from_c(c_ref):
    cm = c_ref[...]
    m = jnp.max(cm, axis=1, keepdims=True)
    e = jnp.exp(cm - m)
    return e / jnp.sum(e, axis=1, keepdims=True)


def _self_term(qs, z):
    return (qs[0:1, 0:1] * z[:, 0:4] + qs[0:1, 1:2] * z[:, 4:8]
            + qs[0:1, 2:3] * z[:, 8:12] + qs[0:1, 3:4] * z[:, 12:16])


def _tc0(x, W1, u1):
    def body(x_ref, w_ref, u_ref, z_ref, a_ref):
        xx = x_ref[...]
        z_ref[...] = _dot(xx, w_ref[...])
        a_ref[...] = _dot(xx, u_ref[...])

    nb = 10
    rb = N // nb
    return pl.pallas_call(
        body,
        grid=(nb,),
        in_specs=[pl.BlockSpec((rb, 128), lambda i: (i, 0)),
                  pl.BlockSpec((128, 16), lambda i: (0, 0)),
                  pl.BlockSpec((128, 4), lambda i: (0, 0))],
        out_specs=[pl.BlockSpec((rb, 16), lambda i: (i, 0)),
                   pl.BlockSpec((rb, 4), lambda i: (i, 0))],
        out_shape=[jax.ShapeDtypeStruct((N, 16), _f32),
                   jax.ShapeDtypeStruct((N, 4), _f32)],
    )(x, W1, u1)


def _tc1(agg, z1, c1, b1, W2, u2):
    def body(agg_ref, z_ref, c_ref, b_ref, w_ref, u_ref,
             z2_ref, a2_ref, deg_ref):
        m = agg_ref[0, :, 0:4] + agg_ref[1, :, 0:4]
        dg = agg_ref[0, :, 4:5] + agg_ref[1, :, 4:5] + 1.0
        qs = _qs_from_c(c_ref)
        z1v = z_ref[...]
        h1 = jnp.maximum((m + _self_term(qs, z1v)) / dg + b_ref[...], 0.0)
        z2_ref[...] = _dot(h1, w_ref[...])
        a2_ref[...] = _dot(h1, u_ref[...])
        deg_ref[...] = dg

    nb = 10
    rb = N // nb
    return pl.pallas_call(
        body,
        grid=(nb,),
        in_specs=[pl.BlockSpec((2, rb, 8), lambda i: (0, i, 0)),
                  pl.BlockSpec((rb, 16), lambda i: (i, 0)),
                  pl.BlockSpec((1, 4), lambda i: (0, 0)),
                  pl.BlockSpec((1, 4), lambda i: (0, 0)),
                  pl.BlockSpec((4, 16), lambda i: (0, 0)),
                  pl.BlockSpec((4, 4), lambda i: (0, 0))],
        out_specs=[pl.BlockSpec((rb, 16), lambda i: (i, 0)),
                   pl.BlockSpec((rb, 4), lambda i: (i, 0)),
                   pl.BlockSpec((rb, 1), lambda i: (i, 0))],
        out_shape=[jax.ShapeDtypeStruct((N, 16), _f32),
                   jax.ShapeDtypeStruct((N, 4), _f32),
                   jax.ShapeDtypeStruct((N, 1), _f32)],
    )(agg, z1, c1, b1, W2, u2)


def _tc_mid(agg, z, deg, c, b, Wn, un):
    def body(agg_ref, z_ref, deg_ref, c_ref, b_ref, w_ref, u_ref,
             zn_ref, an_ref):
        m = agg_ref[0, :, 0:4] + agg_ref[1, :, 0:4]
        qs = _qs_from_c(c_ref)
        hh = jnp.maximum(
            (m + _self_term(qs, z_ref[...])) / deg_ref[...] + b_ref[...], 0.0)
        zn_ref[...] = _dot(hh, w_ref[...])
        an_ref[...] = _dot(hh, u_ref[...])

    nb = 10
    rb = N // nb
    return pl.pallas_call(
        body,
        grid=(nb,),
        in_specs=[pl.BlockSpec((2, rb, 8), lambda i: (0, i, 0)),
                  pl.BlockSpec((rb, 16), lambda i: (i, 0)),
                  pl.BlockSpec((rb, 1), lambda i: (i, 0)),
                  pl.BlockSpec((1, 4), lambda i: (0, 0)),
                  pl.BlockSpec((1, 4), lambda i: (0, 0)),
                  pl.BlockSpec((4, 16), lambda i: (0, 0)),
                  pl.BlockSpec((4, 4), lambda i: (0, 0))],
        out_specs=[pl.BlockSpec((rb, 16), lambda i: (i, 0)),
                   pl.BlockSpec((rb, 4), lambda i: (i, 0))],
        out_shape=[jax.ShapeDtypeStruct((N, 16), _f32),
                   jax.ShapeDtypeStruct((N, 4), _f32)],
    )(agg, z, deg, c, b, Wn, un)


def _tc3(agg, z, deg, c, b, u4):
    def body(agg_ref, z_ref, deg_ref, c_ref, b_ref, u_ref, h3_ref, a4_ref):
        m = agg_ref[0, :, 0:4] + agg_ref[1, :, 0:4]
        qs = _qs_from_c(c_ref)
        hh = jnp.maximum(
            (m + _self_term(qs, z_ref[...])) / deg_ref[...] + b_ref[...], 0.0)
        h3_ref[...] = hh
        a4_ref[...] = _dot(hh, u_ref[...])

    nb = 10
    rb = N // nb
    return pl.pallas_call(
        body,
        grid=(nb,),
        in_specs=[pl.BlockSpec((2, rb, 8), lambda i: (0, i, 0)),
                  pl.BlockSpec((rb, 16), lambda i: (i, 0)),
                  pl.BlockSpec((rb, 1), lambda i: (i, 0)),
                  pl.BlockSpec((1, 4), lambda i: (0, 0)),
                  pl.BlockSpec((1, 4), lambda i: (0, 0)),
                  pl.BlockSpec((4, 4), lambda i: (0, 0))],
        out_specs=[pl.BlockSpec((rb, 4), lambda i: (i, 0)),
                   pl.BlockSpec((rb, 4), lambda i: (i, 0))],
        out_shape=[jax.ShapeDtypeStruct((N, 4), _f32),
                   jax.ShapeDtypeStruct((N, 4), _f32)],
    )(agg, z, deg, c, b, u4)


def _tc4(t, h3, deg, W4r, c4, b4, gamma, beta):
    def body_h4(t_ref, h3_ref, deg_ref, w_ref, c_ref, b_ref, out_ref):
        qs = _qs_from_c(c_ref)
        h3v = h3_ref[...]
        tt = t_ref[0] + t_ref[1] + jnp.concatenate(
            [qs[0:1, h:h + 1] * h3v for h in range(4)], axis=1)
        agg = _dot(tt, w_ref[...])
        out_ref[...] = jnp.maximum(agg / deg_ref[...] + b_ref[...], 0.0)

    nb = 10
    rb = N // nb
    h4 = pl.pallas_call(
        body_h4,
        grid=(nb,),
        in_specs=[pl.BlockSpec((2, rb, 16), lambda i: (0, i, 0)),
                  pl.BlockSpec((rb, 4), lambda i: (i, 0)),
                  pl.BlockSpec((rb, 1), lambda i: (i, 0)),
                  pl.BlockSpec((16, 128), lambda i: (0, 0)),
                  pl.BlockSpec((1, 4), lambda i: (0, 0)),
                  pl.BlockSpec((1, 128), lambda i: (0, 0))],
        out_specs=pl.BlockSpec((rb, 128), lambda i: (i, 0)),
        out_shape=jax.ShapeDtypeStruct((N, D_OUT), _f32),
    )(t, h3, deg, W4r, c4, b4)

    def body_bn(h4_ref, g_ref, be_ref, out_ref):
        hv = h4_ref[...]
        mean = jnp.mean(hv, axis=0, keepdims=True)
        var = jnp.mean((hv - mean) ** 2, axis=0, keepdims=True)
        out_ref[...] = (hv - mean) / jnp.sqrt(var + 1e-5) * g_ref[...] \
            + be_ref[...]

    return pl.pallas_call(
        body_bn,
        out_shape=jax.ShapeDtypeStruct((N, D_OUT), _f32),
    )(h4, gamma, beta)


def kernel(x, edge_index, W1, u1, c1, b1, W2, u2, c2, b2, W3, u3, c3, b3,
           W4, u4, c4, b4, gamma, beta):
    src = edge_index[0]
    dst = edge_index[1]
    cvec = [jnp.tile(c[:, None], (1, 16)).astype(_f32) for c in (c1, c2, c3, c4)]
    zeros8 = jnp.zeros((N, 8), _f32)
    zeros16 = jnp.zeros((N, 16), _f32)
    W4r = W4.reshape(4, 4, 128).transpose(1, 0, 2).reshape(16, 128)
    zform = _sc_zform()

    z1, a1 = _tc0(x, W1, u1)
    agg1 = zform(src, dst, a1.reshape(-1), cvec[0], z1, zeros8)
    z2, a2, deg = _tc1(agg1, z1, c1.reshape(1, 4), b1.reshape(1, 4), W2, u2)
    agg2 = zform(src, dst, a2.reshape(-1), cvec[1], z2, zeros8)
    z3, a3 = _tc_mid(agg2, z2, deg, c2.reshape(1, 4), b2.reshape(1, 4), W3, u3)
    agg3 = zform(src, dst, a3.reshape(-1), cvec[2], z3, zeros8)
    h3, a4 = _tc3(agg3, z3, deg, c3.reshape(1, 4), b3.reshape(1, 4), u4)
    t = _sc_l4()(src, dst, a4.reshape(-1), cvec[3], h3.reshape(-1), zeros16)
    return _tc4(t, h3, deg, W4r, c4.reshape(1, 4), b4.reshape(1, 128),
                gamma.reshape(1, 128), beta.reshape(1, 128))
